# baseline (speedup 1.0000x reference)
import functools

import jax
import jax.numpy as jnp
from jax import lax
from jax.experimental import pallas as pl
from jax.experimental.pallas import tpu as pltpu
from jax.experimental.pallas import tpu_sc as plsc

_LANES = 16
_NW = 32
_BATCH = 16384
_NF = 16
_BPW = _BATCH // _NW
_NBLK = _BPW // _LANES
_NCHUNK = 4
_CHUNK = _BPW // _NCHUNK


def _shuf(v, idx):
    return v.at[idx].get(mode="promise_in_bounds")


@functools.partial(
    pl.kernel,
    mesh=plsc.VectorSubcoreMesh(core_axis_name="c", subcore_axis_name="s"),
    out_type=jax.ShapeDtypeStruct((_BATCH,), jnp.float32),
    compiler_params=pltpu.CompilerParams(use_tc_tiling_on_sc=False),
    scratch_types=[
        pltpu.VMEM((2 * _BPW,), jnp.int32),
        pltpu.VMEM((_NCHUNK, _CHUNK), jnp.int32),
        pltpu.VMEM((_NCHUNK, _CHUNK), jnp.int32),
        pltpu.VMEM((_BPW, _NF), jnp.float32),
        pltpu.VMEM((_BPW, _NF), jnp.float32),
        pltpu.VMEM((_BPW,), jnp.float32),
        pltpu.SemaphoreType.DMA,
    ],
)
def _sc_dot(x_hbm, uf_hbm, mf_hbm, out_hbm,
            x_v, uidx_v, midx_v, u_v, m_v, o_v, sem):
    wid = lax.axis_index("s") * 2 + lax.axis_index("c")
    base = wid * _BPW

    pltpu.sync_copy(x_hbm.at[pl.ds(2 * base, 2 * _BPW)], x_v)

    iota = lax.broadcasted_iota(jnp.int32, (_LANES,), 0)
    even = (iota * 2) % _LANES
    odd = even + 1
    lo_half = iota < (_LANES // 2)

    for j in range(_NBLK):
        a = x_v[pl.ds(2 * j * _LANES, _LANES)]
        b = x_v[pl.ds(2 * j * _LANES + _LANES, _LANES)]
        u_i = jnp.where(lo_half, _shuf(a, even), _shuf(b, even))
        m_i = jnp.where(lo_half, _shuf(a, odd), _shuf(b, odd))
        uidx_v[j // 8, pl.ds((j % 8) * _LANES, _LANES)] = u_i
        midx_v[j // 8, pl.ds((j % 8) * _LANES, _LANES)] = m_i

    copies = []
    for k in range(_NCHUNK):
        copies.append(pltpu.async_copy(
            uf_hbm.at[uidx_v.at[k]], u_v.at[pl.ds(k * _CHUNK, _CHUNK), :], sem))
        copies.append(pltpu.async_copy(
            mf_hbm.at[midx_v.at[k]], m_v.at[pl.ds(k * _CHUNK, _CHUNK), :], sem))
    for c in copies:
        c.wait()

    bfly = [iota ^ (1 << s) for s in range(4)]

    def blk(j, carry):
        row0 = j * _LANES
        acc = jnp.zeros((_LANES,), jnp.float32)
        for r in range(_LANES):
            p = u_v[row0 + r, :] * m_v[row0 + r, :]
            for idx in bfly:
                p = p + _shuf(p, idx)
            acc = jnp.where(iota == r, p, acc)
        o_v[pl.ds(row0, _LANES)] = 5.5 / (1.0 + jnp.exp(-acc))
        return carry

    lax.fori_loop(0, _NBLK, blk, 0)

    pltpu.sync_copy(o_v, out_hbm.at[pl.ds(base, _BPW)])


def kernel(x, user_factors, movie_factors):
    return _sc_dot(x.reshape(-1), user_factors, movie_factors)
